# baseline (speedup 1.0000x reference)
import functools

import jax
import jax.numpy as jnp
from jax import lax
from jax.experimental import pallas as pl
from jax.experimental.pallas import tpu as pltpu
from jax.experimental.pallas import tpu_sc as plsc

NC = 2
NS = 16
NW = NC * NS

N = 100000
DP = 128
NP = 100352
DCH = NP // NW
TOTAL = 500000
TCH = TOTAL // NW
TCHP = 15632
HTCH = TCHP // 2
HLEN = 3 * HTCH + 8
NK = 2
NPH = NP // NK
DCHK = NPH // NW
SUB = 224
NSUB = DCHK // SUB

HS = 64
BR = 1568

_mesh = functools.partial(plsc.VectorSubcoreMesh,
                          core_axis_name="c", subcore_axis_name="s")
_SC_PARAMS = pltpu.CompilerParams(needs_layout_passes=False,
                                  use_tc_tiling_on_sc=False)


def _wid():
    return lax.axis_index("s") * NC + lax.axis_index("c")


def _a1_body(i1d_ref, neg1_ref, partial_ref, pw, dbuf):
    wid = _wid()
    pltpu.sync_copy(neg1_ref, pw)
    base = wid * TCH
    lanes = lax.iota(jnp.int32, 16)

    halves = ((0, 488, 23440, False), (7808, 489, 23480, True))
    for o, nv, ln, is_last in halves:
        start3 = 3 * (base + o)
        s8 = (start3 // 8) * 8
        skew = start3 - s8
        if is_last:
            last_s = (3 * ((NW - 1) * TCH + o) // 8) * 8
            last_ln = 3 * TOTAL - last_s

            @pl.when(wid == NW - 1)
            def _():
                pltpu.sync_copy(i1d_ref.at[pl.ds(last_s, last_ln)],
                                dbuf.at[pl.ds(0, last_ln)])

            @pl.when(wid < NW - 1)
            def _():
                pltpu.sync_copy(
                    i1d_ref.at[pl.ds(pl.multiple_of(s8, 8), ln)],
                    dbuf.at[pl.ds(0, ln)])
        else:
            pltpu.sync_copy(i1d_ref.at[pl.ds(pl.multiple_of(s8, 8), ln)],
                            dbuf.at[pl.ds(0, ln)])

        def vreg_body(k, _):
            t = k * 16 + lanes
            rank = base + o + k * 16 + lanes
            dv_raw = plsc.load_gather(dbuf, [skew + 3 * t + 2])
            dv = jnp.where(rank < TOTAL, dv_raw, NP - 1)
            plsc.store_scatter(pw, [dv], rank)
            pend = plsc.load_gather(pw, [dv]) < rank

            def cond(p):
                return jnp.max(jnp.where(p, 1, 0)) > 0

            def body(p):
                plsc.store_scatter(pw, [dv], rank, mask=p)
                return plsc.load_gather(pw, [dv]) < rank

            lax.while_loop(cond, body, pend)
            return 0

        lax.fori_loop(0, nv, vreg_body, 0)

    pltpu.sync_copy(pw, partial_ref.at[pl.ds(wid * NP, NP)])


def _a1(i1d, neg1):
    return pl.kernel(
        _a1_body,
        out_type=jax.ShapeDtypeStruct((NW * NP,), jnp.int32),
        mesh=_mesh(),
        compiler_params=_SC_PARAMS,
        scratch_types=[
            pltpu.VMEM((NP,), jnp.int32),
            pltpu.VMEM((HLEN + 24,), jnp.int32),
        ],
    )(i1d, neg1)


def _a2b_body(off, partial_ref, i1d_ref, feat_ref, x_ref, m2_ref,
              fl_ref, fr_ref, xb_ref,
              pbuf, rbuf, rbufb, lbuf, ribuf, mbx, xmbuf, fb0, fb1,
              semf0, semf1, semx):
    wid = _wid()
    g0 = off + wid * DCHK
    r0 = wid * DCHK
    pltpu.sync_copy(partial_ref.at[:, pl.ds(g0, DCHK)], pbuf)

    lanes = lax.iota(jnp.int32, 16)

    def merge_v(j, _):
        dsj = pl.ds(j * 16, 16)
        a = pbuf[0, dsj]
        for s in range(1, NW):
            p = pbuf[s, dsj]
            a = jnp.where(p >= 0, p, a)
        r3 = 3 * jnp.clip(a, 0, TOTAL - 1)
        rbuf[dsj] = r3
        rbufb[dsj] = r3 + 1
        d = g0 + j * 16 + lanes
        mbx[dsj] = jnp.where(a >= 0, 512, 0) + (d & 511)
        return 0

    lax.fori_loop(0, DCHK // 16, merge_v, 0)

    d0 = pltpu.async_copy(i1d_ref.at[rbuf], lbuf, semf0)
    d1 = pltpu.async_copy(i1d_ref.at[rbufb], ribuf, semf1)
    d0.wait()
    d1.wait()

    allrows = pl.ds(r0, DCHK)
    for idx, xcol in ((lbuf, 0), (ribuf, 8), (mbx, 16)):
        src = x_ref if xcol < 16 else m2_ref
        dx = pltpu.async_copy(src.at[idx], xmbuf, semx)
        dx.wait()
        pltpu.sync_copy(xmbuf, xb_ref.at[allrows, pl.ds(xcol, 8)])

    for idxbuf, out_ref in ((lbuf, fl_ref), (ribuf, fr_ref)):
        def gather(c, buf, sem):
            idx = idxbuf.at[pl.ds(c * SUB, SUB)]
            pltpu.async_copy(feat_ref.at[idx], buf, sem)

        def drain(buf, sem):
            pltpu.make_async_copy(feat_ref.at[pl.ds(0, SUB)], buf, sem).wait()

        def write(c, buf):
            pltpu.sync_copy(buf, out_ref.at[pl.ds(r0 + c * SUB, SUB)])

        gather(0, fb0, semf0)
        gather(1, fb1, semf1)

        def pipe(g, _):
            c = 2 * g
            drain(fb0, semf0)
            write(c, fb0)
            gather(c + 2, fb0, semf0)
            drain(fb1, semf1)
            write(c + 1, fb1)
            gather(c + 3, fb1, semf1)
            return 0

        lax.fori_loop(0, (NSUB - 3) // 2, pipe, 0)
        drain(fb0, semf0)
        write(NSUB - 3, fb0)
        gather(NSUB - 1, fb0, semf0)
        drain(fb1, semf1)
        write(NSUB - 2, fb1)
        drain(fb0, semf0)
        write(NSUB - 1, fb0)


def _a2b(k, partial, i1d, feature, x, m2):
    nin = x.shape[1]
    return pl.kernel(
        functools.partial(_a2b_body, k * NPH),
        out_type=[
            jax.ShapeDtypeStruct((NPH, DP), jnp.float32),
            jax.ShapeDtypeStruct((NPH, DP), jnp.float32),
            jax.ShapeDtypeStruct((NPH, DP), jnp.float32),
        ],
        mesh=_mesh(),
        compiler_params=_SC_PARAMS,
        scratch_types=[
            pltpu.VMEM((NW, DCHK), jnp.int32),
            pltpu.VMEM((DCHK,), jnp.int32),
            pltpu.VMEM((DCHK,), jnp.int32),
            pltpu.VMEM((DCHK,), jnp.int32),
            pltpu.VMEM((DCHK,), jnp.int32),
            pltpu.VMEM((DCHK,), jnp.int32),
            pltpu.VMEM((DCHK, nin), jnp.float32),
            pltpu.VMEM((SUB, DP), jnp.float32),
            pltpu.VMEM((SUB, DP), jnp.float32),
            pltpu.SemaphoreType.DMA,
            pltpu.SemaphoreType.DMA,
            pltpu.SemaphoreType.DMA,
        ],
        name=f"a2b_chunk{k}",
    )(partial, i1d, feature, x, m2)


def _sigmoid(x):
    return 0.5 * jnp.tanh(0.5 * x) + 0.5


def _lstm_block(xb_ref, fl_ref, fr_ref, f_ref, wih_ref, whh_ref, bias_ref,
                *rest):
    out_ref = rest[-1]
    wih = wih_ref[...]
    whh = whh_ref[...]
    bias = bias_ref[...]
    xb = xb_ref[...]

    def branch(x, fea):
        gates = (
            jax.lax.dot_general(x, wih, (((1,), (0,)), ((), ())),
                                preferred_element_type=jnp.float32)
            + jax.lax.dot_general(fea[:, :HS], whh, (((1,), (0,)), ((), ())),
                                  preferred_element_type=jnp.float32)
            + bias)
        i = _sigmoid(gates[:, 0:64])
        f = _sigmoid(gates[:, 64:128])
        gg = jnp.tanh(gates[:, 128:192])
        o = _sigmoid(gates[:, 192:256])
        c = fea[:, HS:2 * HS]
        c_new = f * c + i * gg
        h_new = o * jnp.tanh(c_new)
        return h_new, c_new

    h_l, c_l = branch(xb[:, 0:8], fl_ref[...])
    h_r, c_r = branch(xb[:, 8:16], fr_ref[...])
    h = h_l + h_r
    c = c_l + c_r
    mask = xb[:, 16:17] > 0.5
    out_ref[...] = jnp.where(mask, jnp.concatenate([h, c], axis=1),
                             f_ref[...])


def _lstm_pass(k, XB, FL, FR, Feature, WihT, WhhT, bias2, prev):
    n, d = Feature.shape
    nin = WihT.shape[0]
    koff = k * (NPH // BR)
    in_specs = [
        pl.BlockSpec((BR, DP), lambda i: (i, 0)),
        pl.BlockSpec((BR, d), lambda i: (i, 0)),
        pl.BlockSpec((BR, d), lambda i: (i, 0)),
        pl.BlockSpec((BR, d), lambda i: (i + koff, 0)),
        pl.BlockSpec((nin, 256), lambda i: (0, 0)),
        pl.BlockSpec((HS, 256), lambda i: (0, 0)),
        pl.BlockSpec((1, 256), lambda i: (0, 0)),
    ]
    args = [XB, FL, FR, Feature, WihT, WhhT, bias2]
    aliases = {}
    if prev is not None:
        in_specs.append(pl.BlockSpec(memory_space=pl.ANY))
        args.append(prev)
        aliases = {7: 0}
    return pl.pallas_call(
        _lstm_block,
        grid=(NPH // BR,),
        in_specs=in_specs,
        out_specs=pl.BlockSpec((BR, d), lambda i: (i + koff, 0)),
        out_shape=jax.ShapeDtypeStruct((n, d), jnp.float32),
        input_output_aliases=aliases,
    )(*args)


def kernel(X, Feature, I_list, W_ih, W_hh, b_ih, b_hh):
    n, d = Feature.shape
    nlvl, _, ni, _ = I_list.shape
    total = nlvl * ni

    i1d = I_list.reshape(3 * total)
    neg1 = jnp.full((NP,), -1, jnp.int32)
    m2 = jnp.repeat(jnp.array([0.0, 1.0], jnp.float32), 512)[:, None]
    m2 = jnp.broadcast_to(m2, (1024, X.shape[1])).copy()

    partial = _a1(i1d, neg1)

    WihT = W_ih.T
    WhhT = W_hh.T
    bias2 = (b_ih + b_hh).reshape(1, 256)

    partial2 = partial.reshape(NW, NP)
    out = None
    for k in range(NK):
        FL, FR, XB = _a2b(k, partial2, i1d, Feature, X, m2)
        out = _lstm_pass(k, XB, FL, FR, Feature, WihT, WhhT, bias2, out)
    return out


# speedup vs baseline: 4.0634x; 4.0634x over previous
import functools

import jax
import jax.numpy as jnp
from jax import lax
from jax.experimental import pallas as pl
from jax.experimental.pallas import tpu as pltpu
from jax.experimental.pallas import tpu_sc as plsc

NC = 2
NS = 16
NW = NC * NS

N = 100000
DP = 128
NP = 100352
DCH = NP // NW
TOTAL = 500000
TCH = TOTAL // NW
TCHP = 15632
TOTALP = NW * TCHP
NK = 2
NPH = NP // NK
DCHK = NPH // NW
SUB = 224
NSUB = DCHK // SUB

HS = 64
BR = 1568

_mesh = functools.partial(plsc.VectorSubcoreMesh,
                          core_axis_name="c", subcore_axis_name="s")
_SC_PARAMS = pltpu.CompilerParams(needs_layout_passes=False,
                                  use_tc_tiling_on_sc=False)


def _wid():
    return lax.axis_index("s") * NC + lax.axis_index("c")


def _a1_body(dest_ref, neg1_ref, partial_ref, pw, dbuf):
    wid = _wid()
    pltpu.sync_copy(neg1_ref, pw)
    pltpu.sync_copy(dest_ref.at[pl.ds(wid * TCHP, TCHP)], dbuf)
    base = wid * TCHP
    lanes = lax.iota(jnp.int32, 16)

    def vreg_body(k, _):
        dv = dbuf[pl.ds(k * 16, 16)]
        rank = base + k * 16 + lanes
        plsc.store_scatter(pw, [dv], rank)
        pend = plsc.load_gather(pw, [dv]) < rank

        def cond(p):
            return jnp.max(jnp.where(p, 1, 0)) > 0

        def body(p):
            plsc.store_scatter(pw, [dv], rank, mask=p)
            return plsc.load_gather(pw, [dv]) < rank

        lax.while_loop(cond, body, pend)
        return 0

    lax.fori_loop(0, TCHP // 16, vreg_body, 0)
    pltpu.sync_copy(pw, partial_ref.at[pl.ds(wid * NP, NP)])


def _a1(dest1d, neg1):
    return pl.kernel(
        _a1_body,
        out_type=jax.ShapeDtypeStruct((NW * NP,), jnp.int32),
        mesh=_mesh(),
        compiler_params=_SC_PARAMS,
        scratch_types=[
            pltpu.VMEM((NP,), jnp.int32),
            pltpu.VMEM((TCHP,), jnp.int32),
        ],
    )(dest1d, neg1)


def _a2b_body(off, partial_ref, i0_ref, i1_ref, feat_ref, x_ref, m2_ref,
              fl_ref, fr_ref, xb_ref,
              pbuf, rbuf, lbuf, ribuf, mbx, xmbuf, fb0, fb1,
              semf0, semf1, semx):
    wid = _wid()
    g0 = off + wid * DCHK
    r0 = wid * DCHK
    pltpu.sync_copy(partial_ref.at[:, pl.ds(g0, DCHK)], pbuf)

    lanes = lax.iota(jnp.int32, 16)

    def merge_v(j, _):
        dsj = pl.ds(j * 16, 16)
        a = pbuf[0, dsj]
        for s in range(1, NW):
            p = pbuf[s, dsj]
            a = jnp.where(p >= 0, p, a)
        rbuf[dsj] = jnp.clip(a, 0, TOTALP - 1)
        d = g0 + j * 16 + lanes
        mbx[dsj] = jnp.where(a >= 0, 512, 0) + (d & 511)
        return 0

    lax.fori_loop(0, DCHK // 16, merge_v, 0)

    d0 = pltpu.async_copy(i0_ref.at[rbuf], lbuf, semf0)
    d1 = pltpu.async_copy(i1_ref.at[rbuf], ribuf, semf1)
    d0.wait()
    d1.wait()

    allrows = pl.ds(r0, DCHK)
    for idx, xcol in ((lbuf, 0), (ribuf, 8), (mbx, 16)):
        src = x_ref if xcol < 16 else m2_ref
        dx = pltpu.async_copy(src.at[idx], xmbuf, semx)
        dx.wait()
        pltpu.sync_copy(xmbuf, xb_ref.at[allrows, pl.ds(xcol, 8)])

    for idxbuf, out_ref in ((lbuf, fl_ref), (ribuf, fr_ref)):
        def gather(c, buf, sem):
            idx = idxbuf.at[pl.ds(c * SUB, SUB)]
            pltpu.async_copy(feat_ref.at[idx], buf, sem)

        def drain(buf, sem):
            pltpu.make_async_copy(feat_ref.at[pl.ds(0, SUB)], buf, sem).wait()

        def write(c, buf):
            pltpu.sync_copy(buf, out_ref.at[pl.ds(r0 + c * SUB, SUB)])

        gather(0, fb0, semf0)
        gather(1, fb1, semf1)

        def pipe(g, _):
            c = 2 * g
            drain(fb0, semf0)
            write(c, fb0)
            gather(c + 2, fb0, semf0)
            drain(fb1, semf1)
            write(c + 1, fb1)
            gather(c + 3, fb1, semf1)
            return 0

        lax.fori_loop(0, (NSUB - 3) // 2, pipe, 0)
        drain(fb0, semf0)
        write(NSUB - 3, fb0)
        gather(NSUB - 1, fb0, semf0)
        drain(fb1, semf1)
        write(NSUB - 2, fb1)
        drain(fb0, semf0)
        write(NSUB - 1, fb0)


def _a2b(k, partial, i0p, i1p, feature, x, m2):
    nin = x.shape[1]
    return pl.kernel(
        functools.partial(_a2b_body, k * NPH),
        out_type=[
            jax.ShapeDtypeStruct((NPH, DP), jnp.float32),
            jax.ShapeDtypeStruct((NPH, DP), jnp.float32),
            jax.ShapeDtypeStruct((NPH, DP), jnp.float32),
        ],
        mesh=_mesh(),
        compiler_params=_SC_PARAMS,
        scratch_types=[
            pltpu.VMEM((NW, DCHK), jnp.int32),
            pltpu.VMEM((DCHK,), jnp.int32),
            pltpu.VMEM((DCHK,), jnp.int32),
            pltpu.VMEM((DCHK,), jnp.int32),
            pltpu.VMEM((DCHK,), jnp.int32),
            pltpu.VMEM((DCHK, nin), jnp.float32),
            pltpu.VMEM((SUB, DP), jnp.float32),
            pltpu.VMEM((SUB, DP), jnp.float32),
            pltpu.SemaphoreType.DMA,
            pltpu.SemaphoreType.DMA,
            pltpu.SemaphoreType.DMA,
        ],
        name=f"a2b_chunk{k}",
    )(partial, i0p, i1p, feature, x, m2)


def _sigmoid(x):
    return 0.5 * jnp.tanh(0.5 * x) + 0.5


def _lstm_block(xb_ref, fl_ref, fr_ref, f_ref, wih_ref, whh_ref, bias_ref,
                *rest):
    out_ref = rest[-1]
    wih = wih_ref[...]
    whh = whh_ref[...]
    bias = bias_ref[...]
    xb = xb_ref[...]

    def branch(x, fea):
        gates = (
            jax.lax.dot_general(x, wih, (((1,), (0,)), ((), ())),
                                preferred_element_type=jnp.float32)
            + jax.lax.dot_general(fea[:, :HS], whh, (((1,), (0,)), ((), ())),
                                  preferred_element_type=jnp.float32)
            + bias)
        i = _sigmoid(gates[:, 0:64])
        f = _sigmoid(gates[:, 64:128])
        gg = jnp.tanh(gates[:, 128:192])
        o = _sigmoid(gates[:, 192:256])
        c = fea[:, HS:2 * HS]
        c_new = f * c + i * gg
        h_new = o * jnp.tanh(c_new)
        return h_new, c_new

    h_l, c_l = branch(xb[:, 0:8], fl_ref[...])
    h_r, c_r = branch(xb[:, 8:16], fr_ref[...])
    h = h_l + h_r
    c = c_l + c_r
    mask = xb[:, 16:17] > 0.5
    out_ref[...] = jnp.where(mask, jnp.concatenate([h, c], axis=1),
                             f_ref[...])


def _lstm_pass(k, XB, FL, FR, Feature, WihT, WhhT, bias2, prev):
    n, d = Feature.shape
    nin = WihT.shape[0]
    koff = k * (NPH // BR)
    in_specs = [
        pl.BlockSpec((BR, DP), lambda i: (i, 0)),
        pl.BlockSpec((BR, d), lambda i: (i, 0)),
        pl.BlockSpec((BR, d), lambda i: (i, 0)),
        pl.BlockSpec((BR, d), lambda i: (i + koff, 0)),
        pl.BlockSpec((nin, 256), lambda i: (0, 0)),
        pl.BlockSpec((HS, 256), lambda i: (0, 0)),
        pl.BlockSpec((1, 256), lambda i: (0, 0)),
    ]
    args = [XB, FL, FR, Feature, WihT, WhhT, bias2]
    aliases = {}
    if prev is not None:
        in_specs.append(pl.BlockSpec(memory_space=pl.ANY))
        args.append(prev)
        aliases = {7: 0}
    return pl.pallas_call(
        _lstm_block,
        grid=(NPH // BR,),
        in_specs=in_specs,
        out_specs=pl.BlockSpec((BR, d), lambda i: (i + koff, 0)),
        out_shape=jax.ShapeDtypeStruct((n, d), jnp.float32),
        input_output_aliases=aliases,
    )(*args)


def kernel(X, Feature, I_list, W_ih, W_hh, b_ih, b_hh):
    n, d = Feature.shape
    nlvl, _, ni, _ = I_list.shape
    total = nlvl * ni

    If3 = I_list[:, 0].reshape(total, 3)
    dest1d = (jnp.full((NW, TCHP), NP - 1, jnp.int32)
              .at[:, :TCH].set(If3[:, 2].reshape(NW, TCH)).reshape(TOTALP))
    i0p = (jnp.zeros((NW, TCHP), jnp.int32)
           .at[:, :TCH].set(If3[:, 0].reshape(NW, TCH)).reshape(TOTALP))
    i1p = (jnp.zeros((NW, TCHP), jnp.int32)
           .at[:, :TCH].set(If3[:, 1].reshape(NW, TCH)).reshape(TOTALP))
    neg1 = jnp.full((NP,), -1, jnp.int32)
    m2 = jnp.repeat(jnp.array([0.0, 1.0], jnp.float32), 512)[:, None]
    m2 = jnp.broadcast_to(m2, (1024, X.shape[1])).copy()

    partial = _a1(dest1d, neg1)

    WihT = W_ih.T
    WhhT = W_hh.T
    bias2 = (b_ih + b_hh).reshape(1, 256)

    partial2 = partial.reshape(NW, NP)
    out = None
    for k in range(NK):
        FL, FR, XB = _a2b(k, partial2, i0p, i1p, Feature, X, m2)
        out = _lstm_pass(k, XB, FL, FR, Feature, WihT, WhhT, bias2, out)
    return out
